# Initial kernel scaffold; baseline (speedup 1.0000x reference)
import jax
import jax.numpy as jnp
from jax.experimental import pallas as pl


def kernel(features1, edge_index1, edgesAttr1, adjacency1, node2node_features1, W_att, a_att, W_gcn, b_gcn, W_out, b_out):
    raise NotImplementedError("write your pallas kernel here")


# speedup vs baseline: 107.1773x; 107.1773x over previous
import functools

import jax
import jax.numpy as jnp
from jax import lax
from jax.experimental import pallas as pl
from jax.experimental.pallas import tpu as pltpu
from jax.experimental.pallas import tpu_sc as plsc

_N = 10000
_E = 320000
_D = 128
_NSUB = 16
_NPAD = 10240
_EC = _E // _NSUB
_NC = _NPAD // _NSUB
_L = 16


def _rsqrt16(x):
    i = lax.bitcast_convert_type(x, jnp.int32)
    i = jnp.int32(0x5F3759DF) - (i >> 1)
    y = lax.bitcast_convert_type(i, jnp.float32)
    for _ in range(3):
        y = y * (jnp.float32(1.5) - jnp.float32(0.5) * x * y * y)
    return y


def _sc_node_weights(src_hbm, dst_hbm, w_hbm,
                     src_v, dst_v, val_v, node_a, node_b,
                     deg_sp, dis_sp, s_sp):
    wid = lax.axis_index("s")
    ebase = wid * _EC
    nbase = wid * _NC

    pltpu.sync_copy(src_hbm.at[pl.ds(ebase, _EC)], src_v)
    pltpu.sync_copy(dst_hbm.at[pl.ds(ebase, _EC)], dst_v)

    def _fill_ones(i, c):
        val_v[pl.ds(i * _L, _L)] = jnp.full((_L,), 1.0, jnp.float32)
        return c
    lax.fori_loop(0, _EC // _L, _fill_ones, 0)

    def _fill_zeros(i, c):
        node_a[pl.ds(i * _L, _L)] = jnp.zeros((_L,), jnp.float32)
        return c
    lax.fori_loop(0, _NC // _L, _fill_zeros, 0)

    pltpu.sync_copy(val_v.at[pl.ds(0, _NC)], deg_sp.at[pl.ds(nbase, _NC)])
    pltpu.sync_copy(node_a, s_sp.at[pl.ds(nbase, _NC)])
    plsc.subcore_barrier()

    pltpu.sync_copy(val_v, deg_sp.at[dst_v], add=True)
    plsc.subcore_barrier()

    pltpu.sync_copy(deg_sp.at[pl.ds(nbase, _NC)], node_a)

    def _newton(i, c):
        sl = pl.ds(i * _L, _L)
        node_b[sl] = _rsqrt16(node_a[sl])
        return c
    lax.fori_loop(0, _NC // _L, _newton, 0)
    pltpu.sync_copy(node_b, dis_sp.at[pl.ds(nbase, _NC)])
    plsc.subcore_barrier()

    pltpu.sync_copy(dis_sp.at[dst_v], val_v)
    pltpu.sync_copy(val_v, s_sp.at[src_v], add=True)
    plsc.subcore_barrier()

    pltpu.sync_copy(s_sp.at[pl.ds(nbase, _NC)], node_a)

    def _wfin(i, c):
        sl = pl.ds(i * _L, _L)
        d = node_b[sl]
        node_a[sl] = d * (node_a[sl] + d)
        return c
    lax.fori_loop(0, _NC // _L, _wfin, 0)
    pltpu.sync_copy(node_a, w_hbm.at[pl.ds(nbase, _NC)])


_sc_kernel = functools.partial(
    pl.kernel,
    out_type=jax.ShapeDtypeStruct((_NPAD,), jnp.float32),
    mesh=plsc.VectorSubcoreMesh(core_axis_name="c", subcore_axis_name="s",
                                num_cores=1),
    scratch_types=[
        pltpu.VMEM((_EC,), jnp.int32),
        pltpu.VMEM((_EC,), jnp.int32),
        pltpu.VMEM((_EC,), jnp.float32),
        pltpu.VMEM((_NC,), jnp.float32),
        pltpu.VMEM((_NC,), jnp.float32),
        pltpu.VMEM_SHARED((_NPAD,), jnp.float32),
        pltpu.VMEM_SHARED((_NPAD,), jnp.float32),
        pltpu.VMEM_SHARED((_NPAD,), jnp.float32),
    ],
)(_sc_node_weights)


def _tc_body(x_ref, wa_ref, aa_ref, w_ref, wg_ref, bg_ref, wo_ref, bo_ref,
             o_ref):
    x = x_ref[...]
    wh = jnp.dot(x, wa_ref[...], preferred_element_type=jnp.float32)
    e = jnp.dot(wh, aa_ref[...], preferred_element_type=jnp.float32)
    e = jnp.where(e > 0, e, jnp.float32(0.2) * e)
    p = jnp.exp(e - jnp.max(e))
    attn = p * (jnp.float32(1.0) / jnp.sum(p))
    z = attn * wh
    h1 = jnp.where(z > 0, z, jnp.exp(z) - jnp.float32(1.0))
    v = jnp.sum(w_ref[...] * h1, axis=0, keepdims=True)
    g = (jnp.dot(v, wg_ref[...], preferred_element_type=jnp.float32)
         * jnp.float32(1.0 / _N) + bg_ref[...])
    r = (jnp.dot(g, wo_ref[...], preferred_element_type=jnp.float32)
         + bo_ref[...])
    r = r - jnp.max(r)
    pr = jnp.exp(r)
    o_ref[...] = pr * (jnp.float32(1.0) / jnp.sum(pr))


def kernel(features1, edge_index1, edgesAttr1, adjacency1, node2node_features1,
           W_att, a_att, W_gcn, b_gcn, W_out, b_out):
    src = edge_index1[0]
    dst = edge_index1[1]
    w_pad = _sc_kernel(src, dst)
    w = w_pad[:_N].reshape(_N, 1)
    out = pl.pallas_call(
        _tc_body,
        out_shape=jax.ShapeDtypeStruct((1, 2), jnp.float32),
    )(features1, W_att, a_att, w, W_gcn, b_gcn.reshape(1, _D), W_out,
      b_out.reshape(1, 2))
    return out


# speedup vs baseline: 140.0315x; 1.3065x over previous
import functools

import jax
import jax.numpy as jnp
from jax import lax
from jax.experimental import pallas as pl
from jax.experimental.pallas import tpu as pltpu
from jax.experimental.pallas import tpu_sc as plsc

_N = 10000
_E = 320000
_D = 128
_NSUB = 16
_NPAD = 10240
_EC = _E // _NSUB
_NC = _NPAD // _NSUB
_L = 16
_DUMMY = _NPAD - 1


def _rsqrt16(x):
    i = lax.bitcast_convert_type(x, jnp.int32)
    i = jnp.int32(0x5F3759DF) - (i >> 1)
    y = lax.bitcast_convert_type(i, jnp.float32)
    for _ in range(3):
        y = y * (jnp.float32(1.5) - jnp.float32(0.5) * x * y * y)
    return y


def _sc_node_weights(ei_hbm, w_hbm,
                     src_v, dst_v, val_v, node_a, node_b,
                     deg_sp, dis_sp, s_sp):
    wid = lax.axis_index("s")
    ebase = wid * _EC
    nbase = wid * _NC

    pltpu.sync_copy(ei_hbm.at[0, pl.ds(ebase, _EC)], src_v)
    pltpu.sync_copy(ei_hbm.at[1, pl.ds(ebase, _EC)], dst_v)

    def _fill_ones(i, c):
        val_v[pl.ds(i * _L, _L)] = jnp.full((_L,), 1.0, jnp.float32)
        return c
    lax.fori_loop(0, _EC // _L, _fill_ones, 0)

    def _fill_zeros(i, c):
        node_a[pl.ds(i * _L, _L)] = jnp.zeros((_L,), jnp.float32)
        return c
    lax.fori_loop(0, _NC // _L, _fill_zeros, 0)

    pltpu.sync_copy(val_v.at[pl.ds(0, _NC)], deg_sp.at[pl.ds(nbase, _NC)])
    pltpu.sync_copy(node_a, s_sp.at[pl.ds(nbase, _NC)])
    plsc.subcore_barrier()

    pltpu.sync_copy(val_v, deg_sp.at[dst_v], add=True)
    plsc.subcore_barrier()

    pltpu.sync_copy(deg_sp.at[pl.ds(nbase, _NC)], node_a)

    def _newton(i, c):
        sl = pl.ds(i * _L, _L)
        node_b[sl] = _rsqrt16(node_a[sl])
        return c
    lax.fori_loop(0, _NC // _L, _newton, 0)
    pltpu.sync_copy(node_b, dis_sp.at[pl.ds(nbase, _NC)])
    plsc.subcore_barrier()

    pltpu.sync_copy(dis_sp.at[dst_v], val_v)
    pltpu.sync_copy(val_v, s_sp.at[src_v], add=True)
    plsc.subcore_barrier()

    pltpu.sync_copy(s_sp.at[pl.ds(nbase, _NC)], node_a)

    def _wfin(i, c):
        sl = pl.ds(i * _L, _L)
        d = node_b[sl]
        node_a[sl] = d * (node_a[sl] + d)
        return c
    lax.fori_loop(0, _NC // _L, _wfin, 0)
    pltpu.sync_copy(node_a, w_hbm.at[0, pl.ds(nbase, _NC)])


_sc_kernel = functools.partial(
    pl.kernel,
    out_type=jax.ShapeDtypeStruct((1, _NPAD), jnp.float32),
    mesh=plsc.VectorSubcoreMesh(core_axis_name="c", subcore_axis_name="s",
                                num_cores=1),
    compiler_params=pltpu.CompilerParams(use_tc_tiling_on_sc=False),
    scratch_types=[
        pltpu.VMEM((_EC,), jnp.int32),
        pltpu.VMEM((_EC,), jnp.int32),
        pltpu.VMEM((_EC,), jnp.float32),
        pltpu.VMEM((_NC,), jnp.float32),
        pltpu.VMEM((_NC,), jnp.float32),
        pltpu.VMEM_SHARED((_NPAD,), jnp.float32),
        pltpu.VMEM_SHARED((_NPAD,), jnp.float32),
        pltpu.VMEM_SHARED((_NPAD,), jnp.float32),
    ],
)(_sc_node_weights)


def _tc_body(x_ref, wa_ref, aa_ref, w_ref, wg_ref, bg_ref, wo_ref, bo_ref,
             o_ref):
    x = x_ref[...]
    wh = jnp.dot(x, wa_ref[...], preferred_element_type=jnp.float32)
    e = jnp.dot(wh, aa_ref[...], preferred_element_type=jnp.float32)
    e = jnp.where(e > 0, e, jnp.float32(0.2) * e)
    p = jnp.exp(e - jnp.max(e))
    attn = p * (jnp.float32(1.0) / jnp.sum(p))
    z = attn * wh
    h1 = jnp.where(z > 0, z, jnp.exp(z) - jnp.float32(1.0))
    v = jnp.dot(w_ref[...][:, :_N], h1,
                preferred_element_type=jnp.float32)
    g = (jnp.dot(v, wg_ref[...], preferred_element_type=jnp.float32)
         * jnp.float32(1.0 / _N) + bg_ref[...])
    r = (jnp.dot(g, wo_ref[...], preferred_element_type=jnp.float32)
         + bo_ref[...])
    r = r - jnp.max(r)
    pr = jnp.exp(r)
    o_ref[...] = pr * (jnp.float32(1.0) / jnp.sum(pr))


def kernel(features1, edge_index1, edgesAttr1, adjacency1, node2node_features1,
           W_att, a_att, W_gcn, b_gcn, W_out, b_out):
    w_row = _sc_kernel(edge_index1)
    out = pl.pallas_call(
        _tc_body,
        out_shape=jax.ShapeDtypeStruct((1, 2), jnp.float32),
    )(features1, W_att, a_att, w_row, W_gcn, b_gcn.reshape(1, _D), W_out,
      b_out.reshape(1, 2))
    return out
